# baseline (speedup 1.0000x reference)
import functools

import jax
import jax.numpy as jnp
from jax import lax
from jax.experimental import pallas as pl
from jax.experimental.pallas import tpu as pltpu
from jax.experimental.pallas import tpu_sc as plsc

D = 64
B = 16384
_V = 100000

_ROWS_PER_BLOCK = 4096
_GCH = 256


def _sc_gather_one(table, idx):
    info = plsc.get_sparse_core_info()
    nc, ns = info.num_cores, info.num_subcores
    nw = nc * ns
    bpw = B // nw
    nch = bpw // _GCH
    mesh = plsc.VectorSubcoreMesh(core_axis_name="c", subcore_axis_name="s")

    @functools.partial(
        pl.kernel,
        mesh=mesh,
        compiler_params=pltpu.CompilerParams(use_tc_tiling_on_sc=True),
        out_type=jax.ShapeDtypeStruct((B, 2 * D), jnp.float32),
        scratch_types=[
            pltpu.VMEM((bpw,), jnp.int32),
            pltpu.VMEM((_GCH, 2 * D), jnp.float32),
            pltpu.VMEM((_GCH, 2 * D), jnp.float32),
            pltpu.SemaphoreType.DMA,
            pltpu.SemaphoreType.DMA,
        ],
    )
    def gather_kernel(t, i, o, iv, r0, r1, s0, s1):
        wid = lax.axis_index("s") * nc + lax.axis_index("c")
        base = wid * bpw
        bufs = (r0, r1)
        sems = (s0, s1)
        pltpu.sync_copy(i.at[pl.ds(base, bpw)], iv)
        copies = [None] * nch
        copies[0] = pltpu.async_copy(t.at[iv.at[pl.ds(0, _GCH)]], r0, s0)
        for ch in range(nch):
            if ch + 1 < nch:
                copies[ch + 1] = pltpu.async_copy(
                    t.at[iv.at[pl.ds((ch + 1) * _GCH, _GCH)]],
                    bufs[(ch + 1) % 2],
                    sems[(ch + 1) % 2],
                )
            copies[ch].wait()
            pltpu.sync_copy(bufs[ch % 2], o.at[pl.ds(base + ch * _GCH, _GCH)])

    return gather_kernel(table, idx)


def _bind_cosine_body(g1_ref, g2_ref, c_ref, out_ref):
    c = c_ref[...]
    e1 = g1_ref[:, :D]
    e2 = g2_ref[:, :D]
    w1 = jnp.dot(e1, c, preferred_element_type=jnp.float32)
    w2 = jnp.dot(e2, c, preferred_element_type=jnp.float32)
    num = jnp.sum(w1 * w2, axis=-1)
    n1 = jnp.sum(w1 * w1, axis=-1)
    n2 = jnp.sum(w2 * w2, axis=-1)
    out_ref[...] = num / (jnp.sqrt(n1) * jnp.sqrt(n2) + 1e-8)


def _bind_cosine(g1, g2, circ, interpret=False):
    r = _ROWS_PER_BLOCK
    g = B // r
    out = pl.pallas_call(
        _bind_cosine_body,
        grid=(g,),
        in_specs=[
            pl.BlockSpec((r, 2 * D), lambda i: (i, 0)),
            pl.BlockSpec((r, 2 * D), lambda i: (i, 0)),
            pl.BlockSpec((D, D), lambda i: (0, 0)),
        ],
        out_specs=pl.BlockSpec((r,), lambda i: (i,)),
        out_shape=jax.ShapeDtypeStruct((B,), jnp.float32),
        interpret=interpret,
    )(g1, g2, circ)
    return out


def _circulant(d):
    dd = jnp.concatenate([d, d])
    return jnp.stack([lax.slice(dd, (D - j,), (2 * D - j,)) for j in range(D)])


def kernel(source1, source2, dummy_vector, word1, word2):
    i1 = word1.astype(jnp.int32)
    i2 = word2.astype(jnp.int32)
    table1 = jnp.pad(source1, ((0, 0), (0, D)))
    g1 = _sc_gather_one(table1, i1)
    table2 = jnp.pad(source2, ((0, 0), (0, D)))
    g2 = _sc_gather_one(table2, i2)
    return _bind_cosine(g1, g2, _circulant(dummy_vector))


# speedup vs baseline: 1.5892x; 1.5892x over previous
import functools

import jax
import jax.numpy as jnp
from jax import lax
from jax.experimental import pallas as pl
from jax.experimental.pallas import tpu as pltpu
from jax.experimental.pallas import tpu_sc as plsc

D = 64
B = 16384
_V = 100000

_ROWS_PER_BLOCK = 4096
_GCH = 256


def _sc_gather_one(table, idx):
    info = plsc.get_sparse_core_info()
    nc, ns = info.num_cores, info.num_subcores
    nw = nc * ns
    bpw = B // nw
    nch = bpw // _GCH
    mesh = plsc.VectorSubcoreMesh(core_axis_name="c", subcore_axis_name="s")

    @functools.partial(
        pl.kernel,
        mesh=mesh,
        compiler_params=pltpu.CompilerParams(use_tc_tiling_on_sc=True),
        out_type=jax.ShapeDtypeStruct((B, 2 * D), jnp.float32),
        scratch_types=[
            pltpu.VMEM((bpw,), jnp.int32),
            pltpu.VMEM((_GCH, 2 * D), jnp.float32),
            pltpu.VMEM((_GCH, 2 * D), jnp.float32),
            pltpu.SemaphoreType.DMA,
            pltpu.SemaphoreType.DMA,
        ],
    )
    def gather_kernel(t, i, o, iv, r0, r1, s0, s1):
        wid = lax.axis_index("s") * nc + lax.axis_index("c")
        base = wid * bpw
        bufs = (r0, r1)
        sems = (s0, s1)
        pltpu.sync_copy(i.at[pl.ds(base, bpw)], iv)
        copies = [None] * nch
        copies[0] = pltpu.async_copy(t.at[iv.at[pl.ds(0, _GCH)]], r0, s0)
        for ch in range(nch):
            if ch + 1 < nch:
                copies[ch + 1] = pltpu.async_copy(
                    t.at[iv.at[pl.ds((ch + 1) * _GCH, _GCH)]],
                    bufs[(ch + 1) % 2],
                    sems[(ch + 1) % 2],
                )
            copies[ch].wait()
            pltpu.sync_copy(bufs[ch % 2], o.at[pl.ds(base + ch * _GCH, _GCH)])

    return gather_kernel(table, idx)


def _bind_cosine_body(g1_ref, g2_ref, c_ref, out_ref):
    c = c_ref[...]
    e1 = g1_ref[:, :D]
    e2 = g2_ref[:, :D]
    w1 = jnp.dot(e1, c, preferred_element_type=jnp.float32)
    w2 = jnp.dot(e2, c, preferred_element_type=jnp.float32)
    num = jnp.sum(w1 * w2, axis=-1)
    n1 = jnp.sum(w1 * w1, axis=-1)
    n2 = jnp.sum(w2 * w2, axis=-1)
    out_ref[...] = num / (jnp.sqrt(n1) * jnp.sqrt(n2) + 1e-8)


def _bind_cosine(g1, g2, circ, interpret=False):
    r = _ROWS_PER_BLOCK
    g = B // r
    out = pl.pallas_call(
        _bind_cosine_body,
        grid=(g,),
        in_specs=[
            pl.BlockSpec((r, 2 * D), lambda i: (i, 0)),
            pl.BlockSpec((r, 2 * D), lambda i: (i, 0)),
            pl.BlockSpec((D, D), lambda i: (0, 0)),
        ],
        out_specs=pl.BlockSpec((r,), lambda i: (i,)),
        out_shape=jax.ShapeDtypeStruct((B,), jnp.float32),
        interpret=interpret,
    )(g1, g2, circ)
    return out


def _circulant(d):
    dd = jnp.concatenate([d, d])
    return jnp.stack([lax.slice(dd, (D - j,), (2 * D - j,)) for j in range(D)])


def kernel(source1, source2, dummy_vector, word1, word2):
    i1 = word1.astype(jnp.int32)
    i2 = word2.astype(jnp.int32)
    eye = jnp.eye(D, dtype=jnp.float32)
    zero = jnp.zeros((D, D), jnp.float32)
    p = jnp.concatenate([eye, zero], axis=1)
    table1 = jnp.dot(source1, p)
    g1 = _sc_gather_one(table1, i1)
    table2 = jnp.dot(source2, p)
    g2 = _sc_gather_one(table2, i2)
    return _bind_cosine(g1, g2, _circulant(dummy_vector))
